# baseline (speedup 1.0000x reference)
import functools

import jax
import jax.numpy as jnp
from jax import lax
from jax.experimental import pallas as pl
from jax.experimental.pallas import tpu as pltpu
from jax.experimental.pallas import tpu_sc as plsc

_D = 64
_B = 16384
_NC = 2
_NS = 16
_NW = _NC * _NS
_L = 16
_K = 128
_PW = _B // _NW
_NCHUNK = _PW // _K


def _sqrt16(x):
    i = plsc.bitcast(x, jnp.int32)
    y = plsc.bitcast((i >> 1) + jnp.int32(0x1FBD1DF5), jnp.float32)
    y = 0.5 * (y + x / y)
    y = 0.5 * (y + x / y)
    y = 0.5 * (y + x / y)
    return y


def _body(idx_hbm, cls_hbm, rel_hbm, out_hbm,
          i3_a, i0_a, i1_a, i2_a, c_a, r_a, d_a,
          i3_b, i0_b, i1_b, i2_b, c_b, r_b, d_b,
          o_v, sem_a, sem_b):
    wid = lax.axis_index("s") * _NC + lax.axis_index("c")
    base = wid * _PW

    lanes = lax.iota(jnp.int32, _L)
    bufs = [(i3_a, i0_a, i1_a, i2_a, c_a, r_a, d_a, sem_a),
            (i3_b, i0_b, i1_b, i2_b, c_b, r_b, d_b, sem_b)]

    def issue(buf, chunk):
        i3_v, i0_v, i1_v, i2_v, c_v, r_v, d_v, sem = buf
        cb = base + chunk * _K
        pltpu.sync_copy(idx_hbm.at[pl.ds(cb * 3, _K * 3)], i3_v)
        for v in range(_K // _L):
            f16 = (lanes + v * _L) * 3
            i0_v[pl.ds(v * _L, _L)] = plsc.load_gather(i3_v, [f16])
            i1_v[pl.ds(v * _L, _L)] = plsc.load_gather(i3_v, [f16 + 1])
            i2_v[pl.ds(v * _L, _L)] = plsc.load_gather(i3_v, [f16 + 2])
        return (pltpu.async_copy(cls_hbm.at[i0_v], c_v, sem),
                pltpu.async_copy(rel_hbm.at[i1_v], r_v, sem),
                pltpu.async_copy(cls_hbm.at[i2_v], d_v, sem))

    def compute(buf, chunk):
        c_v, r_v, d_v = buf[4], buf[5], buf[6]

        def group_body(g, _):
            rows = lanes + g * _L
            zero = jnp.zeros((_L,), jnp.float32)
            acc_dst = zero
            acc_c = zero
            acc_d = zero
            jlo = lanes
            for j in range(_D):
                jhi = jlo + _D
                c1 = plsc.load_gather(c_v, [rows, jlo])
                c2 = plsc.load_gather(c_v, [rows, jhi])
                d1 = plsc.load_gather(d_v, [rows, jlo])
                d2 = plsc.load_gather(d_v, [rows, jhi])
                r1 = plsc.load_gather(r_v, [rows, jlo])
                r2 = plsc.load_gather(r_v, [rows, jhi])
                t = (jnp.abs(c1 - d1 - r1) + jnp.abs(c2) + jnp.abs(d2)
                     - jnp.abs(r2))
                t = jnp.maximum(t, 0.0)
                acc_dst = acc_dst + t * t
                acc_c = acc_c + c1 * c1
                acc_d = acc_d + d1 * d1
                jlo = jlo + 1
                jlo = jnp.where(jlo == _D, 0, jlo)
            res = (_sqrt16(acc_dst)
                   + jnp.abs(_sqrt16(acc_c) - 1.0)
                   + jnp.abs(_sqrt16(acc_d) - 1.0))
            o_v[pl.ds(chunk * _K + g * _L, _L)] = res
            return 0

        lax.fori_loop(0, _K // _L, group_body, 0)

    copies = issue(bufs[0], 0)
    for chunk in range(_NCHUNK):
        if chunk + 1 < _NCHUNK:
            next_copies = issue(bufs[(chunk + 1) % 2], chunk + 1)
        for cp in copies:
            cp.wait()
        compute(bufs[chunk % 2], chunk)
        if chunk + 1 < _NCHUNK:
            copies = next_copies

    pltpu.sync_copy(o_v, out_hbm.at[pl.ds(base, _PW)])


_mesh = plsc.VectorSubcoreMesh(core_axis_name="c", subcore_axis_name="s")

_dbuf = [
    pltpu.VMEM((_K * 3,), jnp.int32),
    pltpu.VMEM((_K,), jnp.int32),
    pltpu.VMEM((_K,), jnp.int32),
    pltpu.VMEM((_K,), jnp.int32),
    pltpu.VMEM((_K, 2 * _D), jnp.float32),
    pltpu.VMEM((_K, 2 * _D), jnp.float32),
    pltpu.VMEM((_K, 2 * _D), jnp.float32),
]

_tb = functools.partial(
    pl.kernel,
    out_type=jax.ShapeDtypeStruct((_B,), jnp.float32),
    mesh=_mesh,
    scratch_types=_dbuf + _dbuf + [
        pltpu.VMEM((_PW,), jnp.float32),
        pltpu.SemaphoreType.DMA,
        pltpu.SemaphoreType.DMA,
    ],
    compiler_params=pltpu.CompilerParams(needs_layout_passes=False),
)(_body)


@jax.jit
def kernel(input, class_emb, rel_emb):
    out = _tb(input.astype(jnp.int32).reshape(-1), class_emb, rel_emb)
    return out.reshape(_B, 1)


# speedup vs baseline: 1.1566x; 1.1566x over previous
import functools

import jax
import jax.numpy as jnp
from jax import lax
from jax.experimental import pallas as pl
from jax.experimental.pallas import tpu as pltpu
from jax.experimental.pallas import tpu_sc as plsc

_D = 64
_B = 16384
_NC = 2
_NS = 16
_NW = _NC * _NS
_L = 16
_K = 128
_PW = _B // _NW
_NCHUNK = _PW // _K


def _sqrt16(x):
    i = plsc.bitcast(x, jnp.int32)
    y = plsc.bitcast((i >> 1) + jnp.int32(0x1FBD1DF5), jnp.float32)
    y = 0.5 * (y + x / y)
    y = 0.5 * (y + x / y)
    y = 0.5 * (y + x / y)
    return y


def _body(i0_hbm, i1_hbm, i2_hbm, cls_hbm, rel_hbm, out_hbm,
          i0_a, i1_a, i2_a, c_a, r_a, d_a,
          i0_b, i1_b, i2_b, c_b, r_b, d_b,
          cls_s, rel_s, o_v, sem_a, sem_b):
    wid = lax.axis_index("s") * _NC + lax.axis_index("c")
    base = wid * _PW

    sid = lax.axis_index("s")
    for tile, src_ref, dst_ref, off, cnt in (
            (0, cls_hbm, cls_s, 0, 512),
            (1, cls_hbm, cls_s, 512, 488),
            (2, rel_hbm, rel_s, 0, 512),
            (3, rel_hbm, rel_s, 512, 488)):
        @pl.when(sid == tile)
        def _(src_ref=src_ref, dst_ref=dst_ref, off=off, cnt=cnt):
            pltpu.sync_copy(src_ref.at[pl.ds(off, cnt)],
                            dst_ref.at[pl.ds(off, cnt)])

    plsc.subcore_barrier()

    lanes = lax.iota(jnp.int32, _L)
    bufs = [(i0_a, i1_a, i2_a, c_a, r_a, d_a, sem_a),
            (i0_b, i1_b, i2_b, c_b, r_b, d_b, sem_b)]

    def issue(buf, chunk):
        i0_v, i1_v, i2_v, c_v, r_v, d_v, sem = buf
        cb = base + chunk * _K
        pltpu.sync_copy(i0_hbm.at[pl.ds(cb, _K)], i0_v)
        pltpu.sync_copy(i1_hbm.at[pl.ds(cb, _K)], i1_v)
        pltpu.sync_copy(i2_hbm.at[pl.ds(cb, _K)], i2_v)
        return (pltpu.async_copy(cls_s.at[i0_v], c_v, sem),
                pltpu.async_copy(rel_s.at[i1_v], r_v, sem),
                pltpu.async_copy(cls_s.at[i2_v], d_v, sem))

    def compute(buf, chunk):
        _, _, _, c_v, r_v, d_v, _ = buf

        def group_body(g, _):
            rows = lanes + g * _L
            zero = jnp.zeros((_L,), jnp.float32)
            acc_dst = zero
            acc_c = zero
            acc_d = zero
            jlo = lanes
            for j in range(_D):
                jhi = jlo + _D
                c1 = plsc.load_gather(c_v, [rows, jlo])
                c2 = plsc.load_gather(c_v, [rows, jhi])
                d1 = plsc.load_gather(d_v, [rows, jlo])
                d2 = plsc.load_gather(d_v, [rows, jhi])
                r1 = plsc.load_gather(r_v, [rows, jlo])
                r2 = plsc.load_gather(r_v, [rows, jhi])
                t = (jnp.abs(c1 - d1 - r1) + jnp.abs(c2) + jnp.abs(d2)
                     - jnp.abs(r2))
                t = jnp.maximum(t, 0.0)
                acc_dst = acc_dst + t * t
                acc_c = acc_c + c1 * c1
                acc_d = acc_d + d1 * d1
                jlo = jlo + 1
                jlo = jnp.where(jlo == _D, 0, jlo)
            res = (_sqrt16(acc_dst)
                   + jnp.abs(_sqrt16(acc_c) - 1.0)
                   + jnp.abs(_sqrt16(acc_d) - 1.0))
            o_v[pl.ds(chunk * _K + g * _L, _L)] = res
            return 0

        lax.fori_loop(0, _K // _L, group_body, 0)

    copies = issue(bufs[0], 0)
    for chunk in range(_NCHUNK):
        if chunk + 1 < _NCHUNK:
            next_copies = issue(bufs[(chunk + 1) % 2], chunk + 1)
        for cp in copies:
            cp.wait()
        compute(bufs[chunk % 2], chunk)
        if chunk + 1 < _NCHUNK:
            copies = next_copies

    pltpu.sync_copy(o_v, out_hbm.at[pl.ds(base, _PW)])


_mesh = plsc.VectorSubcoreMesh(core_axis_name="c", subcore_axis_name="s")

_dbuf = [
    pltpu.VMEM((_K,), jnp.int32),
    pltpu.VMEM((_K,), jnp.int32),
    pltpu.VMEM((_K,), jnp.int32),
    pltpu.VMEM((_K, 2 * _D), jnp.float32),
    pltpu.VMEM((_K, 2 * _D), jnp.float32),
    pltpu.VMEM((_K, 2 * _D), jnp.float32),
]

_tb = functools.partial(
    pl.kernel,
    out_type=jax.ShapeDtypeStruct((_B,), jnp.float32),
    mesh=_mesh,
    scratch_types=_dbuf + _dbuf + [
        pltpu.VMEM_SHARED((1000, 2 * _D), jnp.float32),
        pltpu.VMEM_SHARED((1000, 2 * _D), jnp.float32),
        pltpu.VMEM((_PW,), jnp.float32),
        pltpu.SemaphoreType.DMA,
        pltpu.SemaphoreType.DMA,
    ],
    compiler_params=pltpu.CompilerParams(needs_layout_passes=False),
)(_body)


@jax.jit
def kernel(input, class_emb, rel_emb):
    idx = input.astype(jnp.int32)
    out = _tb(idx[:, 0], idx[:, 1], idx[:, 2], class_emb, rel_emb)
    return out.reshape(_B, 1)


# speedup vs baseline: 1.1742x; 1.0153x over previous
import functools

import jax
import jax.numpy as jnp
from jax import lax
from jax.experimental import pallas as pl
from jax.experimental.pallas import tpu as pltpu
from jax.experimental.pallas import tpu_sc as plsc

_D = 64
_B = 16384
_NC = 2
_NS = 16
_NW = _NC * _NS
_L = 16
_K = 128
_PW = _B // _NW
_NCHUNK = _PW // _K


def _sqrt16(x):
    i = plsc.bitcast(x, jnp.int32)
    y = plsc.bitcast((i >> 1) + jnp.int32(0x1FBD1DF5), jnp.float32)
    y = 0.5 * (y + x / y)
    y = 0.5 * (y + x / y)
    return y


def _body(i0_hbm, i1_hbm, i2_hbm, cls_hbm, rel_hbm, out_hbm,
          i0_a, i1_a, i2_a, c_a, r_a, d_a,
          i0_b, i1_b, i2_b, c_b, r_b, d_b,
          o_v, sem_a, sem_b):
    wid = lax.axis_index("s") * _NC + lax.axis_index("c")
    base = wid * _PW


    lanes = lax.iota(jnp.int32, _L)
    bufs = [(i0_a, i1_a, i2_a, c_a, r_a, d_a, sem_a),
            (i0_b, i1_b, i2_b, c_b, r_b, d_b, sem_b)]

    def issue(buf, chunk):
        i0_v, i1_v, i2_v, c_v, r_v, d_v, sem = buf
        cb = base + chunk * _K
        pltpu.sync_copy(i0_hbm.at[pl.ds(cb, _K)], i0_v)
        pltpu.sync_copy(i1_hbm.at[pl.ds(cb, _K)], i1_v)
        pltpu.sync_copy(i2_hbm.at[pl.ds(cb, _K)], i2_v)
        return (pltpu.async_copy(cls_hbm.at[i0_v], c_v, sem),
                pltpu.async_copy(rel_hbm.at[i1_v], r_v, sem),
                pltpu.async_copy(cls_hbm.at[i2_v], d_v, sem))

    def compute(buf, chunk):
        _, _, _, c_v, r_v, d_v, _ = buf

        def group_body(g, _):
            rows = lanes + g * _L
            zero = jnp.zeros((_L,), jnp.float32)
            acc = [zero] * 6
            for j in range(_D):
                jlo = lanes ^ j
                jhi = jlo + _D
                c1 = plsc.load_gather(c_v, [rows, jlo])
                c2 = plsc.load_gather(c_v, [rows, jhi])
                d1 = plsc.load_gather(d_v, [rows, jlo])
                d2 = plsc.load_gather(d_v, [rows, jhi])
                r1 = plsc.load_gather(r_v, [rows, jlo])
                r2 = plsc.load_gather(r_v, [rows, jhi])
                t = (jnp.abs(c1 - d1 - r1) + jnp.abs(c2) + jnp.abs(d2)
                     - jnp.abs(r2))
                t = jnp.maximum(t, 0.0)
                p = j % 2
                acc[p] = acc[p] + t * t
                acc[2 + p] = acc[2 + p] + c1 * c1
                acc[4 + p] = acc[4 + p] + d1 * d1
            res = (_sqrt16(acc[0] + acc[1])
                   + jnp.abs(_sqrt16(acc[2] + acc[3]) - 1.0)
                   + jnp.abs(_sqrt16(acc[4] + acc[5]) - 1.0))
            o_v[pl.ds(chunk * _K + g * _L, _L)] = res
            return 0

        lax.fori_loop(0, _K // _L, group_body, 0)

    copies = issue(bufs[0], 0)
    for chunk in range(_NCHUNK):
        if chunk + 1 < _NCHUNK:
            next_copies = issue(bufs[(chunk + 1) % 2], chunk + 1)
        for cp in copies:
            cp.wait()
        compute(bufs[chunk % 2], chunk)
        if chunk + 1 < _NCHUNK:
            copies = next_copies

    pltpu.sync_copy(o_v, out_hbm.at[pl.ds(base, _PW)])


_mesh = plsc.VectorSubcoreMesh(core_axis_name="c", subcore_axis_name="s")

_dbuf = [
    pltpu.VMEM((_K,), jnp.int32),
    pltpu.VMEM((_K,), jnp.int32),
    pltpu.VMEM((_K,), jnp.int32),
    pltpu.VMEM((_K, 2 * _D), jnp.float32),
    pltpu.VMEM((_K, 2 * _D), jnp.float32),
    pltpu.VMEM((_K, 2 * _D), jnp.float32),
]

_tb = functools.partial(
    pl.kernel,
    out_type=jax.ShapeDtypeStruct((_B,), jnp.float32),
    mesh=_mesh,
    scratch_types=_dbuf + _dbuf + [
        pltpu.VMEM((_PW,), jnp.float32),
        pltpu.SemaphoreType.DMA,
        pltpu.SemaphoreType.DMA,
    ],
    compiler_params=pltpu.CompilerParams(needs_layout_passes=False),
)(_body)


@jax.jit
def kernel(input, class_emb, rel_emb):
    idx = input.astype(jnp.int32)
    out = _tb(idx[:, 0], idx[:, 1], idx[:, 2], class_emb, rel_emb)
    return out.reshape(_B, 1)
